# baseline (speedup 1.0000x reference)
import functools

import jax
import jax.numpy as jnp
from jax.experimental import pallas as pl
from jax.experimental.pallas import tpu as pltpu

N_POINTS = 16384
N1 = 819
N2 = 40
K = 32
R1 = 0.3
R2 = 1.0


def _fps_body(n_samples, n_valid, x_ref, y_ref, z_ref, inds_ref):
    x = x_ref[...]
    y = y_ref[...]
    z = z_ref[...]
    shape = x.shape
    rows = jax.lax.broadcasted_iota(jnp.int32, shape, 0)
    cols = jax.lax.broadcasted_iota(jnp.int32, shape, 1)
    flat = rows * shape[1] + cols

    px = x[0, 0]
    py = y[0, 0]
    pz = z[0, 0]
    d = (x - px) ** 2 + (y - py) ** 2 + (z - pz) ** 2
    if n_valid < shape[0] * shape[1]:
        d = jnp.where(flat >= n_valid, jnp.float32(-jnp.inf), d)
    inds_ref[0] = 0

    def body(i, d):
        m = jnp.max(d)
        idx = jnp.min(jnp.where(d == m, flat, jnp.int32(2**30)))
        inds_ref[i] = idx
        onehot = (flat == idx).astype(jnp.float32)
        px = jnp.sum(x * onehot)
        py = jnp.sum(y * onehot)
        pz = jnp.sum(z * onehot)
        d = jnp.minimum(d, (x - px) ** 2 + (y - py) ** 2 + (z - pz) ** 2)
        return d

    jax.lax.fori_loop(1, n_samples, body, d)


def _fps(points, n_samples):
    n = points.shape[0]
    pad = (-n) % 1024
    sp = jnp.pad(points, ((0, pad), (0, 0)))
    sp_T = sp.T.reshape(3, -1, 128)
    return pl.pallas_call(
        functools.partial(_fps_body, n_samples, n),
        out_shape=jax.ShapeDtypeStruct((n_samples,), jnp.int32),
        in_specs=[
            pl.BlockSpec(memory_space=pltpu.MemorySpace.VMEM),
            pl.BlockSpec(memory_space=pltpu.MemorySpace.VMEM),
            pl.BlockSpec(memory_space=pltpu.MemorySpace.VMEM),
        ],
        out_specs=pl.BlockSpec(memory_space=pltpu.MemorySpace.SMEM),
    )(sp_T[0], sp_T[1], sp_T[2])


_QB = 8


def _topk_body(x_ref, y_ref, z_ref, q_ref, ids_ref, vals_ref):
    n = x_ref.shape[1]
    qx = q_ref[:, 0:1]
    qy = q_ref[:, 1:2]
    qz = q_ref[:, 2:3]
    d2 = (x_ref[...] - qx) ** 2 + (y_ref[...] - qy) ** 2 + (z_ref[...] - qz) ** 2
    flat = jax.lax.broadcasted_iota(jnp.int32, (_QB, n), 1)
    kcol = jax.lax.broadcasted_iota(jnp.int32, (_QB, K), 1)
    vals0 = jnp.zeros((_QB, K), jnp.float32)
    ids0 = jnp.zeros((_QB, K), jnp.int32)

    def body(r, carry):
        d2, vals, ids = carry
        m = jnp.min(d2, axis=1, keepdims=True)
        idx = jnp.min(jnp.where(d2 == m, flat, jnp.int32(2**30)),
                      axis=1, keepdims=True)
        vals = jnp.where(kcol == r, m, vals)
        ids = jnp.where(kcol == r, idx, ids)
        d2 = jnp.where(flat == idx, jnp.float32(jnp.inf), d2)
        return d2, vals, ids

    _, vals, ids = jax.lax.fori_loop(0, K, body, (d2, vals0, ids0))
    ids_ref[...] = ids
    vals_ref[...] = vals


_G = 128
_QBB = 32


def _fold_min(v, w):
    while w > _G:
        h = w // 2
        v = jnp.minimum(v[:, :h], v[:, h:])
        w = h
    return jnp.min(v, axis=1, keepdims=True)


def _lex_min(cand, candf):
    m = _fold_min(cand, cand.shape[1])
    iv = jnp.where(cand == m, candf, jnp.int32(2**30))
    while iv.shape[1] > _G:
        h = iv.shape[1] // 2
        iv = jnp.minimum(iv[:, :h], iv[:, h:])
    idx = jnp.min(iv, axis=1, keepdims=True)
    return m, idx


def _topk_big_body(x_ref, y_ref, z_ref, q_ref, ids_ref, vals_ref,
                   d2_ref, cand_ref, candf_ref):
    n = x_ref.shape[1]
    ng = n // _G
    qx = q_ref[:, 0:1]
    qy = q_ref[:, 1:2]
    qz = q_ref[:, 2:3]
    d2 = (x_ref[...] - qx) ** 2 + (y_ref[...] - qy) ** 2 + (z_ref[...] - qz) ** 2
    d2_ref[...] = d2

    gm = jnp.concatenate(
        [jnp.min(d2[:, g * _G:(g + 1) * _G], axis=1, keepdims=True)
         for g in range(ng)], axis=1)
    giota = jax.lax.broadcasted_iota(jnp.int32, (_QBB, ng), 1)
    scol = jax.lax.broadcasted_iota(jnp.int32, (_QBB, K), 1)

    def selbody(r, carry):
        gm, gl = carry
        m = jnp.min(gm, axis=1, keepdims=True)
        g = jnp.min(jnp.where(gm == m, giota, jnp.int32(ng)),
                    axis=1, keepdims=True)
        gl = jnp.where(scol == r, g, gl)
        gm = jnp.where(giota == g, jnp.float32(jnp.inf), gm)
        return gm, gl

    _, gl = jax.lax.fori_loop(
        0, K, selbody, (gm, jnp.zeros((_QBB, K), jnp.int32)))

    lane = jax.lax.broadcasted_iota(jnp.int32, (1, _G), 1)
    for r in range(K):
        for q in range(_QBB):
            start = gl[q, r] * _G
            cand_ref[pl.ds(q, 1), pl.ds(r * _G, _G)] = (
                d2_ref[pl.ds(q, 1), pl.ds(start, _G)])
            candf_ref[pl.ds(q, 1), pl.ds(r * _G, _G)] = start + lane

    cand0 = cand_ref[...]
    candf = candf_ref[...]
    kcol = jax.lax.broadcasted_iota(jnp.int32, (_QBB, K), 1)

    def exbody(r, carry):
        cand, vals, ids = carry
        m, idx = _lex_min(cand, candf)
        vals = jnp.where(kcol == r, m, vals)
        ids = jnp.where(kcol == r, idx, ids)
        cand = jnp.where(candf == idx, jnp.float32(jnp.inf), cand)
        return cand, vals, ids

    _, vals, ids = jax.lax.fori_loop(
        0, K, exbody, (cand0, jnp.zeros((_QBB, K), jnp.float32),
                       jnp.zeros((_QBB, K), jnp.int32)))
    ids_ref[...] = ids
    vals_ref[...] = vals


def _topk_big(points, samples):
    s = samples.shape[0]
    spad = (-s) % _QBB
    sp = jnp.pad(samples, ((0, spad), (0, 0)))
    nq = sp.shape[0]
    n = points.shape[0]
    planes = jnp.broadcast_to(points.T[:, None, :], (3, _QBB, n))
    grid = nq // _QBB
    ids, vals = pl.pallas_call(
        _topk_big_body,
        grid=(grid,),
        in_specs=[
            pl.BlockSpec((_QBB, n), lambda i: (0, 0)),
            pl.BlockSpec((_QBB, n), lambda i: (0, 0)),
            pl.BlockSpec((_QBB, n), lambda i: (0, 0)),
            pl.BlockSpec((_QBB, 3), lambda i: (i, 0)),
        ],
        out_specs=[
            pl.BlockSpec((_QBB, K), lambda i: (i, 0)),
            pl.BlockSpec((_QBB, K), lambda i: (i, 0)),
        ],
        out_shape=[
            jax.ShapeDtypeStruct((nq, K), jnp.int32),
            jax.ShapeDtypeStruct((nq, K), jnp.float32),
        ],
        scratch_shapes=[
            pltpu.VMEM((_QBB, n), jnp.float32),
            pltpu.VMEM((_QBB, K * _G), jnp.float32),
            pltpu.VMEM((_QBB, K * _G), jnp.int32),
        ],
    )(planes[0], planes[1], planes[2], sp)
    return ids[:s], vals[:s]


def _topk(points, samples):
    s = samples.shape[0]
    spad = (-s) % _QB
    sp = jnp.pad(samples, ((0, spad), (0, 0)))
    nq = sp.shape[0]
    npad = (-points.shape[0]) % 128
    pp = jnp.pad(points, ((0, npad), (0, 0)), constant_values=1e6)
    n = pp.shape[0]
    planes = jnp.broadcast_to(pp.T[:, None, :], (3, _QB, n))
    grid = nq // _QB
    ids, vals = pl.pallas_call(
        _topk_body,
        grid=(grid,),
        in_specs=[
            pl.BlockSpec((_QB, n), lambda i: (0, 0)),
            pl.BlockSpec((_QB, n), lambda i: (0, 0)),
            pl.BlockSpec((_QB, n), lambda i: (0, 0)),
            pl.BlockSpec((_QB, 3), lambda i: (i, 0)),
        ],
        out_specs=[
            pl.BlockSpec((_QB, K), lambda i: (i, 0)),
            pl.BlockSpec((_QB, K), lambda i: (i, 0)),
        ],
        out_shape=[
            jax.ShapeDtypeStruct((nq, K), jnp.int32),
            jax.ShapeDtypeStruct((nq, K), jnp.float32),
        ],
    )(planes[0], planes[1], planes[2], sp)
    return ids[:s], vals[:s]


def kernel(points, W1, b1, W2, b2, W3, b3, W4, b4,
           Wd1, bd1, Wd2, bd2, Wd3, bd3, Wd4, bd4):
    s_inds = _fps(points, N1)
    samples = points[s_inds]

    ids1, dv1 = _topk_big(points, samples)
    id1 = ids1.reshape(-1)
    v1 = (dv1 <= R1 * R1).reshape(-1)
    rad_points = points[id1]
    midpoints = jnp.repeat(samples, K, axis=0)
    relative = (rad_points - midpoints) / R1 * v1[:, None].astype(points.dtype)

    s2_inds = _fps(samples, N2)
    samples2 = samples[s2_inds]

    ids2, dv2 = _topk(samples, samples2)
    id2 = ids2.reshape(-1)
    v2 = (dv2 <= R2 * R2).reshape(-1)
    rad2_points = samples[id2]
    midpoints2 = jnp.repeat(samples2, K, axis=0)
    relative2 = (rad2_points - midpoints2) / R2 * v2[:, None].astype(points.dtype)

    feats = _encoder1(relative, W1, b1[None], W2, b2[None])

    fg = feats[id2]
    encoding = _encoder2(relative2, fg, W3[:3], W3[3:], b3[None], W4, b4[None])

    bd3m = jax.scipy.linalg.block_diag(*([Wd3] * 20))
    bd4m = jax.scipy.linalg.block_diag(*([Wd4] * 20))
    dec60, dec1200 = _decoder(encoding, feats, Wd1[:256], Wd1[256:], bd1[None],
                              Wd2, bd2[None], bd3m, jnp.tile(bd3, 20)[None],
                              bd4m, jnp.tile(bd4, 20)[None])

    midpoints_out = (jnp.repeat(samples2, 20, axis=0) + dec60.reshape(800, 3)) * R2
    points_out = (jnp.repeat(midpoints_out, 20, axis=0)
                  + dec1200.reshape(16000, 3)) * R1
    return points_out


_CB = 32
_PREC = jax.lax.Precision.HIGHEST


def _enc1_body(rel_ref, w1_ref, b1_ref, w2_ref, b2_ref, out_ref):
    h = jnp.maximum(jnp.dot(rel_ref[...], w1_ref[...], precision=_PREC)
                    + b1_ref[...], 0.0)
    h = jnp.maximum(jnp.dot(h, w2_ref[...], precision=_PREC)
                    + b2_ref[...], 0.0)
    for c in range(_CB):
        out_ref[pl.ds(c, 1), :] = jnp.max(h[c * K:(c + 1) * K, :],
                                          axis=0, keepdims=True)


def _encoder1(relative, W1, b1, W2, b2):
    nc_pad = 832
    rel = jnp.zeros((nc_pad * K, 3), relative.dtype).at[:relative.shape[0]].set(relative)
    grid = nc_pad // _CB
    return pl.pallas_call(
        _enc1_body,
        grid=(grid,),
        in_specs=[
            pl.BlockSpec((_CB * K, 3), lambda i: (i, 0)),
            pl.BlockSpec((3, 64), lambda i: (0, 0)),
            pl.BlockSpec((1, 64), lambda i: (0, 0)),
            pl.BlockSpec((64, 128), lambda i: (0, 0)),
            pl.BlockSpec((1, 128), lambda i: (0, 0)),
        ],
        out_specs=pl.BlockSpec((_CB, 128), lambda i: (i, 0)),
        out_shape=jax.ShapeDtypeStruct((nc_pad, 128), jnp.float32),
    )(rel, W1, b1, W2, b2)


def _enc2_body(rel_ref, fg_ref, w3a_ref, w3b_ref, b3_ref, w4_ref, b4_ref,
               out_ref):
    h = jnp.dot(rel_ref[...], w3a_ref[...], precision=_PREC)
    h = h + jnp.dot(fg_ref[...], w3b_ref[...], precision=_PREC)
    h = jnp.maximum(h + b3_ref[...], 0.0)
    h = jnp.maximum(jnp.dot(h, w4_ref[...], precision=_PREC)
                    + b4_ref[...], 0.0)
    for c in range(N2):
        out_ref[pl.ds(c, 1), :] = jnp.max(h[c * K:(c + 1) * K, :],
                                          axis=0, keepdims=True)


def _encoder2(relative2, fg, W3a, W3b, b3, W4, b4):
    return pl.pallas_call(
        _enc2_body,
        out_shape=jax.ShapeDtypeStruct((N2, 256), jnp.float32),
    )(relative2, fg, W3a, W3b, b3, W4, b4)


def _dec_body(n_valid, enc_ref, feats_ref, wd1a_ref, wd1b_ref, bd1_ref,
              wd2_ref, bd2_ref, bd3m_ref, bd3_ref, bd4m_ref, bd4_ref,
              o60_ref, o1200_ref):
    feats = feats_ref[...]
    rows = jax.lax.broadcasted_iota(jnp.int32, feats.shape, 0)
    gfeat = jnp.max(jnp.where(rows < n_valid, feats, -jnp.inf),
                    axis=0, keepdims=True)
    d1 = jnp.dot(enc_ref[...], wd1a_ref[...], precision=_PREC)
    d1 = d1 + jnp.dot(gfeat, wd1b_ref[...], precision=_PREC)
    d1 = jnp.maximum(d1 + bd1_ref[...], 0.0)
    mid = jnp.dot(d1, wd2_ref[...], precision=_PREC) + bd2_ref[...]
    o60_ref[...] = jnp.dot(mid, bd3m_ref[...], precision=_PREC) + bd3_ref[...]
    o1200_ref[...] = jnp.dot(mid, bd4m_ref[...], precision=_PREC) + bd4_ref[...]


def _decoder(encoding, feats, Wd1a, Wd1b, bd1, Wd2, bd2, bd3m, bd3t,
             bd4m, bd4t):
    return pl.pallas_call(
        functools.partial(_dec_body, N1),
        out_shape=[
            jax.ShapeDtypeStruct((N2, 60), jnp.float32),
            jax.ShapeDtypeStruct((N2, 1200), jnp.float32),
        ],
    )(encoding, feats, Wd1a, Wd1b, bd1, Wd2, bd2, bd3m, bd3t, bd4m, bd4t)


# speedup vs baseline: 1.0089x; 1.0089x over previous
import functools

import jax
import jax.numpy as jnp
from jax.experimental import pallas as pl
from jax.experimental.pallas import tpu as pltpu

N_POINTS = 16384
N1 = 819
N2 = 40
K = 32
R1 = 0.3
R2 = 1.0


def _fps_body(n_samples, n_valid, x_ref, y_ref, z_ref, inds_ref):
    x = x_ref[...]
    y = y_ref[...]
    z = z_ref[...]
    shape = x.shape
    rows = jax.lax.broadcasted_iota(jnp.int32, shape, 0)
    cols = jax.lax.broadcasted_iota(jnp.int32, shape, 1)
    flat = rows * shape[1] + cols

    px = x[0, 0]
    py = y[0, 0]
    pz = z[0, 0]
    d = (x - px) ** 2 + (y - py) ** 2 + (z - pz) ** 2
    if n_valid < shape[0] * shape[1]:
        d = jnp.where(flat >= n_valid, jnp.float32(-jnp.inf), d)
    inds_ref[0] = 0
    lane = jax.lax.broadcasted_iota(jnp.int32, (1, shape[1]), 1)

    def body(i, carry):
        d, m = carry
        idx = jnp.min(jnp.where(d == m, flat, jnp.int32(2**30)))
        inds_ref[i] = idx
        r = idx // shape[1]
        c = idx % shape[1]
        cm = lane == c
        px = jnp.sum(jnp.where(cm, x_ref[pl.ds(r, 1), :], 0.0))
        py = jnp.sum(jnp.where(cm, y_ref[pl.ds(r, 1), :], 0.0))
        pz = jnp.sum(jnp.where(cm, z_ref[pl.ds(r, 1), :], 0.0))
        d = jnp.minimum(d, (x - px) ** 2 + (y - py) ** 2 + (z - pz) ** 2)
        return d, jnp.max(d)

    jax.lax.fori_loop(1, n_samples, body, (d, jnp.max(d)))


def _fps(points, n_samples):
    n = points.shape[0]
    pad = (-n) % 1024
    sp = jnp.pad(points, ((0, pad), (0, 0)))
    sp_T = sp.T.reshape(3, -1, 128)
    return pl.pallas_call(
        functools.partial(_fps_body, n_samples, n),
        out_shape=jax.ShapeDtypeStruct((n_samples,), jnp.int32),
        in_specs=[
            pl.BlockSpec(memory_space=pltpu.MemorySpace.VMEM),
            pl.BlockSpec(memory_space=pltpu.MemorySpace.VMEM),
            pl.BlockSpec(memory_space=pltpu.MemorySpace.VMEM),
        ],
        out_specs=pl.BlockSpec(memory_space=pltpu.MemorySpace.SMEM),
    )(sp_T[0], sp_T[1], sp_T[2])


_QB = 8


def _topk_body(x_ref, y_ref, z_ref, q_ref, ids_ref, vals_ref):
    n = x_ref.shape[1]
    qx = q_ref[:, 0:1]
    qy = q_ref[:, 1:2]
    qz = q_ref[:, 2:3]
    d2 = (x_ref[...] - qx) ** 2 + (y_ref[...] - qy) ** 2 + (z_ref[...] - qz) ** 2
    flat = jax.lax.broadcasted_iota(jnp.int32, (_QB, n), 1)
    kcol = jax.lax.broadcasted_iota(jnp.int32, (_QB, K), 1)
    vals0 = jnp.zeros((_QB, K), jnp.float32)
    ids0 = jnp.zeros((_QB, K), jnp.int32)

    def body(r, carry):
        d2, vals, ids = carry
        m = jnp.min(d2, axis=1, keepdims=True)
        idx = jnp.min(jnp.where(d2 == m, flat, jnp.int32(2**30)),
                      axis=1, keepdims=True)
        vals = jnp.where(kcol == r, m, vals)
        ids = jnp.where(kcol == r, idx, ids)
        d2 = jnp.where(flat == idx, jnp.float32(jnp.inf), d2)
        return d2, vals, ids

    _, vals, ids = jax.lax.fori_loop(0, K, body, (d2, vals0, ids0))
    ids_ref[...] = ids
    vals_ref[...] = vals


_G = 128
_QBB = 32


def _fold_min(v, w):
    while w > _G:
        h = w // 2
        v = jnp.minimum(v[:, :h], v[:, h:])
        w = h
    return jnp.min(v, axis=1, keepdims=True)


def _lex_min(cand, candf):
    m = _fold_min(cand, cand.shape[1])
    iv = jnp.where(cand == m, candf, jnp.int32(2**30))
    while iv.shape[1] > _G:
        h = iv.shape[1] // 2
        iv = jnp.minimum(iv[:, :h], iv[:, h:])
    idx = jnp.min(iv, axis=1, keepdims=True)
    return m, idx


def _topk_big_body(x_ref, y_ref, z_ref, q_ref, ids_ref, vals_ref,
                   d2_ref, cand_ref, candf_ref):
    n = x_ref.shape[1]
    ng = n // _G
    qx = q_ref[:, 0:1]
    qy = q_ref[:, 1:2]
    qz = q_ref[:, 2:3]
    d2 = (x_ref[...] - qx) ** 2 + (y_ref[...] - qy) ** 2 + (z_ref[...] - qz) ** 2
    d2_ref[...] = d2

    gm = jnp.concatenate(
        [jnp.min(d2[:, g * _G:(g + 1) * _G], axis=1, keepdims=True)
         for g in range(ng)], axis=1)
    giota = jax.lax.broadcasted_iota(jnp.int32, (_QBB, ng), 1)
    scol = jax.lax.broadcasted_iota(jnp.int32, (_QBB, K), 1)

    def selbody(r, carry):
        gm, gl = carry
        m = jnp.min(gm, axis=1, keepdims=True)
        g = jnp.min(jnp.where(gm == m, giota, jnp.int32(ng)),
                    axis=1, keepdims=True)
        gl = jnp.where(scol == r, g, gl)
        gm = jnp.where(giota == g, jnp.float32(jnp.inf), gm)
        return gm, gl

    _, gl = jax.lax.fori_loop(
        0, K, selbody, (gm, jnp.zeros((_QBB, K), jnp.int32)))

    lane = jax.lax.broadcasted_iota(jnp.int32, (1, _G), 1)
    for r in range(K):
        for q in range(_QBB):
            start = gl[q, r] * _G
            cand_ref[pl.ds(q, 1), pl.ds(r * _G, _G)] = (
                d2_ref[pl.ds(q, 1), pl.ds(start, _G)])
            candf_ref[pl.ds(q, 1), pl.ds(r * _G, _G)] = start + lane

    cand0 = cand_ref[...]
    candf = candf_ref[...]
    kcol = jax.lax.broadcasted_iota(jnp.int32, (_QBB, K), 1)

    def exbody(r, carry):
        cand, vals, ids = carry
        m, idx = _lex_min(cand, candf)
        vals = jnp.where(kcol == r, m, vals)
        ids = jnp.where(kcol == r, idx, ids)
        cand = jnp.where(candf == idx, jnp.float32(jnp.inf), cand)
        return cand, vals, ids

    _, vals, ids = jax.lax.fori_loop(
        0, K, exbody, (cand0, jnp.zeros((_QBB, K), jnp.float32),
                       jnp.zeros((_QBB, K), jnp.int32)))
    ids_ref[...] = ids
    vals_ref[...] = vals


def _topk_big(points, samples):
    s = samples.shape[0]
    spad = (-s) % _QBB
    sp = jnp.pad(samples, ((0, spad), (0, 0)))
    nq = sp.shape[0]
    n = points.shape[0]
    planes = jnp.broadcast_to(points.T[:, None, :], (3, _QBB, n))
    grid = nq // _QBB
    ids, vals = pl.pallas_call(
        _topk_big_body,
        grid=(grid,),
        in_specs=[
            pl.BlockSpec((_QBB, n), lambda i: (0, 0)),
            pl.BlockSpec((_QBB, n), lambda i: (0, 0)),
            pl.BlockSpec((_QBB, n), lambda i: (0, 0)),
            pl.BlockSpec((_QBB, 3), lambda i: (i, 0)),
        ],
        out_specs=[
            pl.BlockSpec((_QBB, K), lambda i: (i, 0)),
            pl.BlockSpec((_QBB, K), lambda i: (i, 0)),
        ],
        out_shape=[
            jax.ShapeDtypeStruct((nq, K), jnp.int32),
            jax.ShapeDtypeStruct((nq, K), jnp.float32),
        ],
        scratch_shapes=[
            pltpu.VMEM((_QBB, n), jnp.float32),
            pltpu.VMEM((_QBB, K * _G), jnp.float32),
            pltpu.VMEM((_QBB, K * _G), jnp.int32),
        ],
    )(planes[0], planes[1], planes[2], sp)
    return ids[:s], vals[:s]


def _topk(points, samples):
    s = samples.shape[0]
    spad = (-s) % _QB
    sp = jnp.pad(samples, ((0, spad), (0, 0)))
    nq = sp.shape[0]
    npad = (-points.shape[0]) % 128
    pp = jnp.pad(points, ((0, npad), (0, 0)), constant_values=1e6)
    n = pp.shape[0]
    planes = jnp.broadcast_to(pp.T[:, None, :], (3, _QB, n))
    grid = nq // _QB
    ids, vals = pl.pallas_call(
        _topk_body,
        grid=(grid,),
        in_specs=[
            pl.BlockSpec((_QB, n), lambda i: (0, 0)),
            pl.BlockSpec((_QB, n), lambda i: (0, 0)),
            pl.BlockSpec((_QB, n), lambda i: (0, 0)),
            pl.BlockSpec((_QB, 3), lambda i: (i, 0)),
        ],
        out_specs=[
            pl.BlockSpec((_QB, K), lambda i: (i, 0)),
            pl.BlockSpec((_QB, K), lambda i: (i, 0)),
        ],
        out_shape=[
            jax.ShapeDtypeStruct((nq, K), jnp.int32),
            jax.ShapeDtypeStruct((nq, K), jnp.float32),
        ],
    )(planes[0], planes[1], planes[2], sp)
    return ids[:s], vals[:s]


def kernel(points, W1, b1, W2, b2, W3, b3, W4, b4,
           Wd1, bd1, Wd2, bd2, Wd3, bd3, Wd4, bd4):
    s_inds = _fps(points, N1)
    samples = points[s_inds]

    ids1, dv1 = _topk_big(points, samples)
    id1 = ids1.reshape(-1)
    v1 = (dv1 <= R1 * R1).reshape(-1)
    rad_points = points[id1]
    midpoints = jnp.repeat(samples, K, axis=0)
    relative = (rad_points - midpoints) / R1 * v1[:, None].astype(points.dtype)

    s2_inds = _fps(samples, N2)
    samples2 = samples[s2_inds]

    ids2, dv2 = _topk(samples, samples2)
    id2 = ids2.reshape(-1)
    v2 = (dv2 <= R2 * R2).reshape(-1)
    rad2_points = samples[id2]
    midpoints2 = jnp.repeat(samples2, K, axis=0)
    relative2 = (rad2_points - midpoints2) / R2 * v2[:, None].astype(points.dtype)

    feats = _encoder1(relative, W1, b1[None], W2, b2[None])

    fg = feats[id2]
    encoding = _encoder2(relative2, fg, W3[:3], W3[3:], b3[None], W4, b4[None])

    bd3m = jax.scipy.linalg.block_diag(*([Wd3] * 20))
    bd4m = jax.scipy.linalg.block_diag(*([Wd4] * 20))
    dec60, dec1200 = _decoder(encoding, feats, Wd1[:256], Wd1[256:], bd1[None],
                              Wd2, bd2[None], bd3m, jnp.tile(bd3, 20)[None],
                              bd4m, jnp.tile(bd4, 20)[None])

    midpoints_out = (jnp.repeat(samples2, 20, axis=0) + dec60.reshape(800, 3)) * R2
    points_out = (jnp.repeat(midpoints_out, 20, axis=0)
                  + dec1200.reshape(16000, 3)) * R1
    return points_out


_CB = 32
_PREC = jax.lax.Precision.HIGHEST


def _enc1_body(rel_ref, w1_ref, b1_ref, w2_ref, b2_ref, out_ref):
    h = jnp.maximum(jnp.dot(rel_ref[...], w1_ref[...], precision=_PREC)
                    + b1_ref[...], 0.0)
    h = jnp.maximum(jnp.dot(h, w2_ref[...], precision=_PREC)
                    + b2_ref[...], 0.0)
    for c in range(_CB):
        out_ref[pl.ds(c, 1), :] = jnp.max(h[c * K:(c + 1) * K, :],
                                          axis=0, keepdims=True)


def _encoder1(relative, W1, b1, W2, b2):
    nc_pad = 832
    rel = jnp.zeros((nc_pad * K, 3), relative.dtype).at[:relative.shape[0]].set(relative)
    grid = nc_pad // _CB
    return pl.pallas_call(
        _enc1_body,
        grid=(grid,),
        in_specs=[
            pl.BlockSpec((_CB * K, 3), lambda i: (i, 0)),
            pl.BlockSpec((3, 64), lambda i: (0, 0)),
            pl.BlockSpec((1, 64), lambda i: (0, 0)),
            pl.BlockSpec((64, 128), lambda i: (0, 0)),
            pl.BlockSpec((1, 128), lambda i: (0, 0)),
        ],
        out_specs=pl.BlockSpec((_CB, 128), lambda i: (i, 0)),
        out_shape=jax.ShapeDtypeStruct((nc_pad, 128), jnp.float32),
    )(rel, W1, b1, W2, b2)


def _enc2_body(rel_ref, fg_ref, w3a_ref, w3b_ref, b3_ref, w4_ref, b4_ref,
               out_ref):
    h = jnp.dot(rel_ref[...], w3a_ref[...], precision=_PREC)
    h = h + jnp.dot(fg_ref[...], w3b_ref[...], precision=_PREC)
    h = jnp.maximum(h + b3_ref[...], 0.0)
    h = jnp.maximum(jnp.dot(h, w4_ref[...], precision=_PREC)
                    + b4_ref[...], 0.0)
    for c in range(N2):
        out_ref[pl.ds(c, 1), :] = jnp.max(h[c * K:(c + 1) * K, :],
                                          axis=0, keepdims=True)


def _encoder2(relative2, fg, W3a, W3b, b3, W4, b4):
    return pl.pallas_call(
        _enc2_body,
        out_shape=jax.ShapeDtypeStruct((N2, 256), jnp.float32),
    )(relative2, fg, W3a, W3b, b3, W4, b4)


def _dec_body(n_valid, enc_ref, feats_ref, wd1a_ref, wd1b_ref, bd1_ref,
              wd2_ref, bd2_ref, bd3m_ref, bd3_ref, bd4m_ref, bd4_ref,
              o60_ref, o1200_ref):
    feats = feats_ref[...]
    rows = jax.lax.broadcasted_iota(jnp.int32, feats.shape, 0)
    gfeat = jnp.max(jnp.where(rows < n_valid, feats, -jnp.inf),
                    axis=0, keepdims=True)
    d1 = jnp.dot(enc_ref[...], wd1a_ref[...], precision=_PREC)
    d1 = d1 + jnp.dot(gfeat, wd1b_ref[...], precision=_PREC)
    d1 = jnp.maximum(d1 + bd1_ref[...], 0.0)
    mid = jnp.dot(d1, wd2_ref[...], precision=_PREC) + bd2_ref[...]
    o60_ref[...] = jnp.dot(mid, bd3m_ref[...], precision=_PREC) + bd3_ref[...]
    o1200_ref[...] = jnp.dot(mid, bd4m_ref[...], precision=_PREC) + bd4_ref[...]


def _decoder(encoding, feats, Wd1a, Wd1b, bd1, Wd2, bd2, bd3m, bd3t,
             bd4m, bd4t):
    return pl.pallas_call(
        functools.partial(_dec_body, N1),
        out_shape=[
            jax.ShapeDtypeStruct((N2, 60), jnp.float32),
            jax.ShapeDtypeStruct((N2, 1200), jnp.float32),
        ],
    )(encoding, feats, Wd1a, Wd1b, bd1, Wd2, bd2, bd3m, bd3t, bd4m, bd4t)


# speedup vs baseline: 1.0112x; 1.0022x over previous
import functools

import jax
import jax.numpy as jnp
from jax.experimental import pallas as pl
from jax.experimental.pallas import tpu as pltpu

N_POINTS = 16384
N1 = 819
N2 = 40
K = 32
R1 = 0.3
R2 = 1.0


def _fps_body(n_samples, n_valid, x_ref, y_ref, z_ref, inds_ref):
    x = x_ref[...]
    y = y_ref[...]
    z = z_ref[...]
    shape = x.shape
    rows = jax.lax.broadcasted_iota(jnp.int32, shape, 0)
    cols = jax.lax.broadcasted_iota(jnp.int32, shape, 1)
    flat = rows * shape[1] + cols

    px = x[0, 0]
    py = y[0, 0]
    pz = z[0, 0]
    d = (x - px) ** 2 + (y - py) ** 2 + (z - pz) ** 2
    if n_valid < shape[0] * shape[1]:
        d = jnp.where(flat >= n_valid, jnp.float32(-jnp.inf), d)
    inds_ref[0] = 0
    lane = jax.lax.broadcasted_iota(jnp.int32, (1, shape[1]), 1)

    def body(i, carry):
        d, m = carry
        idx = jnp.min(jnp.where(d == m, flat, jnp.int32(2**30)))
        inds_ref[i] = idx
        r = idx // shape[1]
        c = idx % shape[1]
        cm = lane == c
        px = jnp.sum(jnp.where(cm, x_ref[pl.ds(r, 1), :], 0.0))
        py = jnp.sum(jnp.where(cm, y_ref[pl.ds(r, 1), :], 0.0))
        pz = jnp.sum(jnp.where(cm, z_ref[pl.ds(r, 1), :], 0.0))
        d = jnp.minimum(d, (x - px) ** 2 + (y - py) ** 2 + (z - pz) ** 2)
        return d, jnp.max(d)

    jax.lax.fori_loop(1, n_samples, body, (d, jnp.max(d)))


def _fps(points, n_samples):
    n = points.shape[0]
    pad = (-n) % 1024
    sp = jnp.pad(points, ((0, pad), (0, 0)))
    sp_T = sp.T.reshape(3, -1, 128)
    return pl.pallas_call(
        functools.partial(_fps_body, n_samples, n),
        out_shape=jax.ShapeDtypeStruct((n_samples,), jnp.int32),
        in_specs=[
            pl.BlockSpec(memory_space=pltpu.MemorySpace.VMEM),
            pl.BlockSpec(memory_space=pltpu.MemorySpace.VMEM),
            pl.BlockSpec(memory_space=pltpu.MemorySpace.VMEM),
        ],
        out_specs=pl.BlockSpec(memory_space=pltpu.MemorySpace.SMEM),
    )(sp_T[0], sp_T[1], sp_T[2])


_QB = 8


def _topk_body(x_ref, y_ref, z_ref, q_ref, ids_ref, vals_ref):
    n = x_ref.shape[1]
    qx = q_ref[:, 0:1]
    qy = q_ref[:, 1:2]
    qz = q_ref[:, 2:3]
    d2 = (x_ref[...] - qx) ** 2 + (y_ref[...] - qy) ** 2 + (z_ref[...] - qz) ** 2
    flat = jax.lax.broadcasted_iota(jnp.int32, (_QB, n), 1)
    kcol = jax.lax.broadcasted_iota(jnp.int32, (_QB, K), 1)
    vals0 = jnp.zeros((_QB, K), jnp.float32)
    ids0 = jnp.zeros((_QB, K), jnp.int32)

    def body(r, carry):
        d2, vals, ids = carry
        m = jnp.min(d2, axis=1, keepdims=True)
        idx = jnp.min(jnp.where(d2 == m, flat, jnp.int32(2**30)),
                      axis=1, keepdims=True)
        vals = jnp.where(kcol == r, m, vals)
        ids = jnp.where(kcol == r, idx, ids)
        d2 = jnp.where(flat == idx, jnp.float32(jnp.inf), d2)
        return d2, vals, ids

    _, vals, ids = jax.lax.fori_loop(0, K, body, (d2, vals0, ids0))
    ids_ref[...] = ids
    vals_ref[...] = vals


_G = 128
_QBB = 32


def _fold_min(v, w):
    while w > _G:
        h = w // 2
        v = jnp.minimum(v[:, :h], v[:, h:])
        w = h
    return jnp.min(v, axis=1, keepdims=True)


def _lex_min(cand, candf):
    m = _fold_min(cand, cand.shape[1])
    iv = jnp.where(cand == m, candf, jnp.int32(2**30))
    while iv.shape[1] > _G:
        h = iv.shape[1] // 2
        iv = jnp.minimum(iv[:, :h], iv[:, h:])
    idx = jnp.min(iv, axis=1, keepdims=True)
    return m, idx


def _topk_big_body(x_ref, y_ref, z_ref, q_ref, ids_ref, vals_ref,
                   d2_ref, cand_ref, candf_ref):
    n = x_ref.shape[1]
    ng = n // _G
    qx = q_ref[:, 0:1]
    qy = q_ref[:, 1:2]
    qz = q_ref[:, 2:3]
    d2 = (x_ref[...] - qx) ** 2 + (y_ref[...] - qy) ** 2 + (z_ref[...] - qz) ** 2
    d2_ref[...] = d2

    gm = jnp.concatenate(
        [jnp.min(d2[:, g * _G:(g + 1) * _G], axis=1, keepdims=True)
         for g in range(ng)], axis=1)
    giota = jax.lax.broadcasted_iota(jnp.int32, (_QBB, ng), 1)
    scol = jax.lax.broadcasted_iota(jnp.int32, (_QBB, K), 1)

    def selbody(r, carry):
        gm, gl = carry
        m = jnp.min(gm, axis=1, keepdims=True)
        g = jnp.min(jnp.where(gm == m, giota, jnp.int32(ng)),
                    axis=1, keepdims=True)
        gl = jnp.where(scol == r, g, gl)
        gm = jnp.where(giota == g, jnp.float32(jnp.inf), gm)
        return gm, gl

    _, gl = jax.lax.fori_loop(
        0, K, selbody, (gm, jnp.zeros((_QBB, K), jnp.int32)))

    lane = jax.lax.broadcasted_iota(jnp.int32, (1, _G), 1)
    for r in range(K):
        for q in range(_QBB):
            start = gl[q, r] * _G
            cand_ref[pl.ds(q, 1), pl.ds(r * _G, _G)] = (
                d2_ref[pl.ds(q, 1), pl.ds(start, _G)])
            candf_ref[pl.ds(q, 1), pl.ds(r * _G, _G)] = start + lane

    cand0 = cand_ref[...]
    candf = candf_ref[...]
    kcol = jax.lax.broadcasted_iota(jnp.int32, (_QBB, K), 1)

    def exbody(r, carry):
        cand, vals, ids = carry
        m, idx = _lex_min(cand, candf)
        vals = jnp.where(kcol == r, m, vals)
        ids = jnp.where(kcol == r, idx, ids)
        cand = jnp.where(candf == idx, jnp.float32(jnp.inf), cand)
        return cand, vals, ids

    _, vals, ids = jax.lax.fori_loop(
        0, K, exbody, (cand0, jnp.zeros((_QBB, K), jnp.float32),
                       jnp.zeros((_QBB, K), jnp.int32)))
    ids_ref[...] = ids
    vals_ref[...] = vals


def _topk_big(points, samples):
    s = samples.shape[0]
    spad = (-s) % _QBB
    sp = jnp.pad(samples, ((0, spad), (0, 0)))
    nq = sp.shape[0]
    n = points.shape[0]
    planes = jnp.broadcast_to(points.T[:, None, :], (3, _QBB, n))
    grid = nq // _QBB
    ids, vals = pl.pallas_call(
        _topk_big_body,
        grid=(grid,),
        in_specs=[
            pl.BlockSpec((_QBB, n), lambda i: (0, 0)),
            pl.BlockSpec((_QBB, n), lambda i: (0, 0)),
            pl.BlockSpec((_QBB, n), lambda i: (0, 0)),
            pl.BlockSpec((_QBB, 3), lambda i: (i, 0)),
        ],
        out_specs=[
            pl.BlockSpec((_QBB, K), lambda i: (i, 0)),
            pl.BlockSpec((_QBB, K), lambda i: (i, 0)),
        ],
        out_shape=[
            jax.ShapeDtypeStruct((nq, K), jnp.int32),
            jax.ShapeDtypeStruct((nq, K), jnp.float32),
        ],
        scratch_shapes=[
            pltpu.VMEM((_QBB, n), jnp.float32),
            pltpu.VMEM((_QBB, K * _G), jnp.float32),
            pltpu.VMEM((_QBB, K * _G), jnp.int32),
        ],
    )(planes[0], planes[1], planes[2], sp)
    return ids[:s], vals[:s]


def _topk(points, samples):
    s = samples.shape[0]
    spad = (-s) % _QB
    sp = jnp.pad(samples, ((0, spad), (0, 0)))
    nq = sp.shape[0]
    npad = (-points.shape[0]) % 128
    pp = jnp.pad(points, ((0, npad), (0, 0)), constant_values=1e6)
    n = pp.shape[0]
    planes = jnp.broadcast_to(pp.T[:, None, :], (3, _QB, n))
    grid = nq // _QB
    ids, vals = pl.pallas_call(
        _topk_body,
        grid=(grid,),
        in_specs=[
            pl.BlockSpec((_QB, n), lambda i: (0, 0)),
            pl.BlockSpec((_QB, n), lambda i: (0, 0)),
            pl.BlockSpec((_QB, n), lambda i: (0, 0)),
            pl.BlockSpec((_QB, 3), lambda i: (i, 0)),
        ],
        out_specs=[
            pl.BlockSpec((_QB, K), lambda i: (i, 0)),
            pl.BlockSpec((_QB, K), lambda i: (i, 0)),
        ],
        out_shape=[
            jax.ShapeDtypeStruct((nq, K), jnp.int32),
            jax.ShapeDtypeStruct((nq, K), jnp.float32),
        ],
    )(planes[0], planes[1], planes[2], sp)
    return ids[:s], vals[:s]


def _sc_gather(table, idx):
    from jax import lax
    from jax.experimental.pallas import tpu_sc as plsc

    B = idx.shape[0]
    D = table.shape[1]
    info = plsc.get_sparse_core_info()
    nc, ns = info.num_cores, info.num_subcores
    nw = nc * ns
    b_per_w = B // nw
    mesh = plsc.VectorSubcoreMesh(core_axis_name="c", subcore_axis_name="s")

    @functools.partial(
        pl.kernel, mesh=mesh,
        out_type=jax.ShapeDtypeStruct((B, D), table.dtype),
        scratch_types=[
            pltpu.VMEM((b_per_w,), jnp.int32),
            pltpu.VMEM((b_per_w, D), table.dtype),
            pltpu.SemaphoreType.DMA,
        ],
    )
    def k(table_hbm, idx_hbm, out_hbm, idx_v, rows_v, sem):
        wid = lax.axis_index("s") * nc + lax.axis_index("c")
        base = wid * b_per_w
        pltpu.sync_copy(idx_hbm.at[pl.ds(base, b_per_w)], idx_v)
        pltpu.async_copy(table_hbm.at[idx_v], rows_v, sem).wait()
        pltpu.sync_copy(rows_v, out_hbm.at[pl.ds(base, b_per_w)])

    return k(table, idx)


def kernel(points, W1, b1, W2, b2, W3, b3, W4, b4,
           Wd1, bd1, Wd2, bd2, Wd3, bd3, Wd4, bd4):
    s_inds = _fps(points, N1)
    samples = points[s_inds]

    ids1, dv1 = _topk_big(points, samples)
    id1 = ids1.reshape(-1)
    v1 = (dv1 <= R1 * R1).reshape(-1)
    pts16 = jnp.pad(points, ((0, 0), (0, 125)))
    id1p = jnp.pad(id1, (0, (-id1.shape[0]) % 256))
    rad_points = _sc_gather(pts16, id1p)[:id1.shape[0], :3]
    midpoints = jnp.repeat(samples, K, axis=0)
    relative = (rad_points - midpoints) / R1 * v1[:, None].astype(points.dtype)

    s2_inds = _fps(samples, N2)
    samples2 = samples[s2_inds]

    ids2, dv2 = _topk(samples, samples2)
    id2 = ids2.reshape(-1)
    v2 = (dv2 <= R2 * R2).reshape(-1)
    rad2_points = samples[id2]
    midpoints2 = jnp.repeat(samples2, K, axis=0)
    relative2 = (rad2_points - midpoints2) / R2 * v2[:, None].astype(points.dtype)

    feats = _encoder1(relative, W1, b1[None], W2, b2[None])

    fg = _sc_gather(feats, id2)
    encoding = _encoder2(relative2, fg, W3[:3], W3[3:], b3[None], W4, b4[None])

    bd3m = jax.scipy.linalg.block_diag(*([Wd3] * 20))
    bd4m = jax.scipy.linalg.block_diag(*([Wd4] * 20))
    dec60, dec1200 = _decoder(encoding, feats, Wd1[:256], Wd1[256:], bd1[None],
                              Wd2, bd2[None], bd3m, jnp.tile(bd3, 20)[None],
                              bd4m, jnp.tile(bd4, 20)[None])

    midpoints_out = (jnp.repeat(samples2, 20, axis=0) + dec60.reshape(800, 3)) * R2
    points_out = (jnp.repeat(midpoints_out, 20, axis=0)
                  + dec1200.reshape(16000, 3)) * R1
    return points_out


_CB = 32
_PREC = jax.lax.Precision.HIGHEST


def _enc1_body(rel_ref, w1_ref, b1_ref, w2_ref, b2_ref, out_ref):
    h = jnp.maximum(jnp.dot(rel_ref[...], w1_ref[...], precision=_PREC)
                    + b1_ref[...], 0.0)
    h = jnp.maximum(jnp.dot(h, w2_ref[...], precision=_PREC)
                    + b2_ref[...], 0.0)
    for c in range(_CB):
        out_ref[pl.ds(c, 1), :] = jnp.max(h[c * K:(c + 1) * K, :],
                                          axis=0, keepdims=True)


def _encoder1(relative, W1, b1, W2, b2):
    nc_pad = 832
    rel = jnp.zeros((nc_pad * K, 3), relative.dtype).at[:relative.shape[0]].set(relative)
    grid = nc_pad // _CB
    return pl.pallas_call(
        _enc1_body,
        grid=(grid,),
        in_specs=[
            pl.BlockSpec((_CB * K, 3), lambda i: (i, 0)),
            pl.BlockSpec((3, 64), lambda i: (0, 0)),
            pl.BlockSpec((1, 64), lambda i: (0, 0)),
            pl.BlockSpec((64, 128), lambda i: (0, 0)),
            pl.BlockSpec((1, 128), lambda i: (0, 0)),
        ],
        out_specs=pl.BlockSpec((_CB, 128), lambda i: (i, 0)),
        out_shape=jax.ShapeDtypeStruct((nc_pad, 128), jnp.float32),
    )(rel, W1, b1, W2, b2)


def _enc2_body(rel_ref, fg_ref, w3a_ref, w3b_ref, b3_ref, w4_ref, b4_ref,
               out_ref):
    h = jnp.dot(rel_ref[...], w3a_ref[...], precision=_PREC)
    h = h + jnp.dot(fg_ref[...], w3b_ref[...], precision=_PREC)
    h = jnp.maximum(h + b3_ref[...], 0.0)
    h = jnp.maximum(jnp.dot(h, w4_ref[...], precision=_PREC)
                    + b4_ref[...], 0.0)
    for c in range(N2):
        out_ref[pl.ds(c, 1), :] = jnp.max(h[c * K:(c + 1) * K, :],
                                          axis=0, keepdims=True)


def _encoder2(relative2, fg, W3a, W3b, b3, W4, b4):
    return pl.pallas_call(
        _enc2_body,
        out_shape=jax.ShapeDtypeStruct((N2, 256), jnp.float32),
    )(relative2, fg, W3a, W3b, b3, W4, b4)


def _dec_body(n_valid, enc_ref, feats_ref, wd1a_ref, wd1b_ref, bd1_ref,
              wd2_ref, bd2_ref, bd3m_ref, bd3_ref, bd4m_ref, bd4_ref,
              o60_ref, o1200_ref):
    feats = feats_ref[...]
    rows = jax.lax.broadcasted_iota(jnp.int32, feats.shape, 0)
    gfeat = jnp.max(jnp.where(rows < n_valid, feats, -jnp.inf),
                    axis=0, keepdims=True)
    d1 = jnp.dot(enc_ref[...], wd1a_ref[...], precision=_PREC)
    d1 = d1 + jnp.dot(gfeat, wd1b_ref[...], precision=_PREC)
    d1 = jnp.maximum(d1 + bd1_ref[...], 0.0)
    mid = jnp.dot(d1, wd2_ref[...], precision=_PREC) + bd2_ref[...]
    o60_ref[...] = jnp.dot(mid, bd3m_ref[...], precision=_PREC) + bd3_ref[...]
    o1200_ref[...] = jnp.dot(mid, bd4m_ref[...], precision=_PREC) + bd4_ref[...]


def _decoder(encoding, feats, Wd1a, Wd1b, bd1, Wd2, bd2, bd3m, bd3t,
             bd4m, bd4t):
    return pl.pallas_call(
        functools.partial(_dec_body, N1),
        out_shape=[
            jax.ShapeDtypeStruct((N2, 60), jnp.float32),
            jax.ShapeDtypeStruct((N2, 1200), jnp.float32),
        ],
    )(encoding, feats, Wd1a, Wd1b, bd1, Wd2, bd2, bd3m, bd3t, bd4m, bd4t)


# speedup vs baseline: 1.1259x; 1.1135x over previous
import functools

import jax
import jax.numpy as jnp
from jax.experimental import pallas as pl
from jax.experimental.pallas import tpu as pltpu

N_POINTS = 16384
N1 = 819
N2 = 40
K = 32
R1 = 0.3
R2 = 1.0


def _fps_body(n_samples, n_valid, x_ref, y_ref, z_ref, inds_ref):
    x = x_ref[...]
    y = y_ref[...]
    z = z_ref[...]
    shape = x.shape
    rows = jax.lax.broadcasted_iota(jnp.int32, shape, 0)
    cols = jax.lax.broadcasted_iota(jnp.int32, shape, 1)
    flat = rows * shape[1] + cols

    px = x[0, 0]
    py = y[0, 0]
    pz = z[0, 0]
    d = (x - px) ** 2 + (y - py) ** 2 + (z - pz) ** 2
    if n_valid < shape[0] * shape[1]:
        d = jnp.where(flat >= n_valid, jnp.float32(-jnp.inf), d)
    inds_ref[0] = 0
    lane = jax.lax.broadcasted_iota(jnp.int32, (1, shape[1]), 1)

    def body(i, carry):
        d, m = carry
        idx = jnp.min(jnp.where(d == m, flat, jnp.int32(2**30)))
        inds_ref[i] = idx
        r = idx // shape[1]
        c = idx % shape[1]
        cm = lane == c
        px = jnp.sum(jnp.where(cm, x_ref[pl.ds(r, 1), :], 0.0))
        py = jnp.sum(jnp.where(cm, y_ref[pl.ds(r, 1), :], 0.0))
        pz = jnp.sum(jnp.where(cm, z_ref[pl.ds(r, 1), :], 0.0))
        d = jnp.minimum(d, (x - px) ** 2 + (y - py) ** 2 + (z - pz) ** 2)
        return d, jnp.max(d)

    jax.lax.fori_loop(1, n_samples, body, (d, jnp.max(d)))


def _fps(points, n_samples):
    n = points.shape[0]
    pad = (-n) % 1024
    sp = jnp.pad(points, ((0, pad), (0, 0)))
    sp_T = sp.T.reshape(3, -1, 128)
    return pl.pallas_call(
        functools.partial(_fps_body, n_samples, n),
        out_shape=jax.ShapeDtypeStruct((n_samples,), jnp.int32),
        in_specs=[
            pl.BlockSpec(memory_space=pltpu.MemorySpace.VMEM),
            pl.BlockSpec(memory_space=pltpu.MemorySpace.VMEM),
            pl.BlockSpec(memory_space=pltpu.MemorySpace.VMEM),
        ],
        out_specs=pl.BlockSpec(memory_space=pltpu.MemorySpace.SMEM),
    )(sp_T[0], sp_T[1], sp_T[2])


_QB = 8


def _topk_body(x_ref, y_ref, z_ref, q_ref, ids_ref, vals_ref):
    n = x_ref.shape[1]
    qx = q_ref[:, 0:1]
    qy = q_ref[:, 1:2]
    qz = q_ref[:, 2:3]
    d2 = (x_ref[...] - qx) ** 2 + (y_ref[...] - qy) ** 2 + (z_ref[...] - qz) ** 2
    flat = jax.lax.broadcasted_iota(jnp.int32, (_QB, n), 1)
    kcol = jax.lax.broadcasted_iota(jnp.int32, (_QB, K), 1)
    vals0 = jnp.zeros((_QB, K), jnp.float32)
    ids0 = jnp.zeros((_QB, K), jnp.int32)

    def body(r, carry):
        d2, vals, ids = carry
        m = jnp.min(d2, axis=1, keepdims=True)
        idx = jnp.min(jnp.where(d2 == m, flat, jnp.int32(2**30)),
                      axis=1, keepdims=True)
        vals = jnp.where(kcol == r, m, vals)
        ids = jnp.where(kcol == r, idx, ids)
        d2 = jnp.where(flat == idx, jnp.float32(jnp.inf), d2)
        return d2, vals, ids

    _, vals, ids = jax.lax.fori_loop(0, K, body, (d2, vals0, ids0))
    ids_ref[...] = ids
    vals_ref[...] = vals


_G = 128
_QBB = 64


def _fold_min(v, w):
    while w > _G:
        h = w // 2
        v = jnp.minimum(v[:, :h], v[:, h:])
        w = h
    return jnp.min(v, axis=1, keepdims=True)


def _lex_min(cand, candf):
    m = _fold_min(cand, cand.shape[1])
    iv = jnp.where(cand == m, candf, jnp.int32(2**30))
    while iv.shape[1] > _G:
        h = iv.shape[1] // 2
        iv = jnp.minimum(iv[:, :h], iv[:, h:])
    idx = jnp.min(iv, axis=1, keepdims=True)
    return m, idx


def _topk_big_body(x_ref, y_ref, z_ref, q_ref, ids_ref, vals_ref,
                   d2_ref, cand_ref, candf_ref):
    n = x_ref.shape[1]
    ng = n // _G
    qx = q_ref[:, 0:1]
    qy = q_ref[:, 1:2]
    qz = q_ref[:, 2:3]
    d2 = (x_ref[...] - qx) ** 2 + (y_ref[...] - qy) ** 2 + (z_ref[...] - qz) ** 2
    d2_ref[...] = d2

    gm = jnp.concatenate(
        [jnp.min(d2[:, g * _G:(g + 1) * _G], axis=1, keepdims=True)
         for g in range(ng)], axis=1)
    giota = jax.lax.broadcasted_iota(jnp.int32, (_QBB, ng), 1)
    scol = jax.lax.broadcasted_iota(jnp.int32, (_QBB, K), 1)

    def selbody(r, carry):
        gm, gl = carry
        m = jnp.min(gm, axis=1, keepdims=True)
        g = jnp.min(jnp.where(gm == m, giota, jnp.int32(ng)),
                    axis=1, keepdims=True)
        gl = jnp.where(scol == r, g, gl)
        gm = jnp.where(giota == g, jnp.float32(jnp.inf), gm)
        return gm, gl

    _, gl = jax.lax.fori_loop(
        0, K, selbody, (gm, jnp.zeros((_QBB, K), jnp.int32)))

    lane = jax.lax.broadcasted_iota(jnp.int32, (1, _G), 1)
    for r in range(K):
        for q in range(_QBB):
            start = gl[q, r] * _G
            cand_ref[pl.ds(q, 1), pl.ds(r * _G, _G)] = (
                d2_ref[pl.ds(q, 1), pl.ds(start, _G)])
            candf_ref[pl.ds(q, 1), pl.ds(r * _G, _G)] = start + lane

    cand0 = cand_ref[...]
    candf = candf_ref[...]
    kcol = jax.lax.broadcasted_iota(jnp.int32, (_QBB, K), 1)

    def exbody(r, carry):
        cand, vals, ids = carry
        m, idx = _lex_min(cand, candf)
        vals = jnp.where(kcol == r, m, vals)
        ids = jnp.where(kcol == r, idx, ids)
        cand = jnp.where(candf == idx, jnp.float32(jnp.inf), cand)
        return cand, vals, ids

    _, vals, ids = jax.lax.fori_loop(
        0, K, exbody, (cand0, jnp.zeros((_QBB, K), jnp.float32),
                       jnp.zeros((_QBB, K), jnp.int32)))
    ids_ref[...] = ids
    vals_ref[...] = vals


def _topk_big(points, samples):
    s = samples.shape[0]
    spad = (-s) % _QBB
    sp = jnp.pad(samples, ((0, spad), (0, 0)))
    nq = sp.shape[0]
    n = points.shape[0]
    planes = jnp.broadcast_to(points.T[:, None, :], (3, _QBB, n))
    grid = nq // _QBB
    ids, vals = pl.pallas_call(
        _topk_big_body,
        grid=(grid,),
        in_specs=[
            pl.BlockSpec((_QBB, n), lambda i: (0, 0)),
            pl.BlockSpec((_QBB, n), lambda i: (0, 0)),
            pl.BlockSpec((_QBB, n), lambda i: (0, 0)),
            pl.BlockSpec((_QBB, 3), lambda i: (i, 0)),
        ],
        out_specs=[
            pl.BlockSpec((_QBB, K), lambda i: (i, 0)),
            pl.BlockSpec((_QBB, K), lambda i: (i, 0)),
        ],
        out_shape=[
            jax.ShapeDtypeStruct((nq, K), jnp.int32),
            jax.ShapeDtypeStruct((nq, K), jnp.float32),
        ],
        scratch_shapes=[
            pltpu.VMEM((_QBB, n), jnp.float32),
            pltpu.VMEM((_QBB, K * _G), jnp.float32),
            pltpu.VMEM((_QBB, K * _G), jnp.int32),
        ],
    )(planes[0], planes[1], planes[2], sp)
    return ids[:s], vals[:s]


def _topk(points, samples):
    s = samples.shape[0]
    spad = (-s) % _QB
    sp = jnp.pad(samples, ((0, spad), (0, 0)))
    nq = sp.shape[0]
    npad = (-points.shape[0]) % 128
    pp = jnp.pad(points, ((0, npad), (0, 0)), constant_values=1e6)
    n = pp.shape[0]
    planes = jnp.broadcast_to(pp.T[:, None, :], (3, _QB, n))
    grid = nq // _QB
    ids, vals = pl.pallas_call(
        _topk_body,
        grid=(grid,),
        in_specs=[
            pl.BlockSpec((_QB, n), lambda i: (0, 0)),
            pl.BlockSpec((_QB, n), lambda i: (0, 0)),
            pl.BlockSpec((_QB, n), lambda i: (0, 0)),
            pl.BlockSpec((_QB, 3), lambda i: (i, 0)),
        ],
        out_specs=[
            pl.BlockSpec((_QB, K), lambda i: (i, 0)),
            pl.BlockSpec((_QB, K), lambda i: (i, 0)),
        ],
        out_shape=[
            jax.ShapeDtypeStruct((nq, K), jnp.int32),
            jax.ShapeDtypeStruct((nq, K), jnp.float32),
        ],
    )(planes[0], planes[1], planes[2], sp)
    return ids[:s], vals[:s]


def _sc_gather(table, idx):
    from jax import lax
    from jax.experimental.pallas import tpu_sc as plsc

    B = idx.shape[0]
    D = table.shape[1]
    info = plsc.get_sparse_core_info()
    nc, ns = info.num_cores, info.num_subcores
    nw = nc * ns
    b_per_w = B // nw
    mesh = plsc.VectorSubcoreMesh(core_axis_name="c", subcore_axis_name="s")

    @functools.partial(
        pl.kernel, mesh=mesh,
        out_type=jax.ShapeDtypeStruct((B, D), table.dtype),
        scratch_types=[
            pltpu.VMEM((b_per_w,), jnp.int32),
            pltpu.VMEM((b_per_w, D), table.dtype),
            pltpu.SemaphoreType.DMA,
        ],
    )
    def k(table_hbm, idx_hbm, out_hbm, idx_v, rows_v, sem):
        wid = lax.axis_index("s") * nc + lax.axis_index("c")
        base = wid * b_per_w
        pltpu.sync_copy(idx_hbm.at[pl.ds(base, b_per_w)], idx_v)
        pltpu.async_copy(table_hbm.at[idx_v], rows_v, sem).wait()
        pltpu.sync_copy(rows_v, out_hbm.at[pl.ds(base, b_per_w)])

    return k(table, idx)


def kernel(points, W1, b1, W2, b2, W3, b3, W4, b4,
           Wd1, bd1, Wd2, bd2, Wd3, bd3, Wd4, bd4):
    s_inds = _fps(points, N1)
    samples = points[s_inds]

    ids1, dv1 = _topk_big(points, samples)
    id1 = ids1.reshape(-1)
    v1 = (dv1 <= R1 * R1).reshape(-1)
    pts16 = jnp.pad(points, ((0, 0), (0, 125)))
    id1p = jnp.pad(id1, (0, (-id1.shape[0]) % 256))
    rad_points = _sc_gather(pts16, id1p)[:id1.shape[0], :3]
    midpoints = jnp.repeat(samples, K, axis=0)
    relative = (rad_points - midpoints) / R1 * v1[:, None].astype(points.dtype)

    s2_inds = _fps(samples, N2)
    samples2 = samples[s2_inds]

    ids2, dv2 = _topk(samples, samples2)
    id2 = ids2.reshape(-1)
    v2 = (dv2 <= R2 * R2).reshape(-1)
    rad2_points = samples[id2]
    midpoints2 = jnp.repeat(samples2, K, axis=0)
    relative2 = (rad2_points - midpoints2) / R2 * v2[:, None].astype(points.dtype)

    feats = _encoder1(relative, W1, b1[None], W2, b2[None])

    fg = _sc_gather(feats, id2)
    encoding = _encoder2(relative2, fg, W3[:3], W3[3:], b3[None], W4, b4[None])

    bd3m = jax.scipy.linalg.block_diag(*([Wd3] * 20))
    bd4m = jax.scipy.linalg.block_diag(*([Wd4] * 20))
    dec60, dec1200 = _decoder(encoding, feats, Wd1[:256], Wd1[256:], bd1[None],
                              Wd2, bd2[None], bd3m, jnp.tile(bd3, 20)[None],
                              bd4m, jnp.tile(bd4, 20)[None])

    midpoints_out = (jnp.repeat(samples2, 20, axis=0) + dec60.reshape(800, 3)) * R2
    points_out = (jnp.repeat(midpoints_out, 20, axis=0)
                  + dec1200.reshape(16000, 3)) * R1
    return points_out


_CB = 32
_PREC = jax.lax.Precision.HIGHEST


def _enc1_body(rel_ref, w1_ref, b1_ref, w2_ref, b2_ref, out_ref):
    h = jnp.maximum(jnp.dot(rel_ref[...], w1_ref[...], precision=_PREC)
                    + b1_ref[...], 0.0)
    h = jnp.maximum(jnp.dot(h, w2_ref[...], precision=_PREC)
                    + b2_ref[...], 0.0)
    for c in range(_CB):
        out_ref[pl.ds(c, 1), :] = jnp.max(h[c * K:(c + 1) * K, :],
                                          axis=0, keepdims=True)


def _encoder1(relative, W1, b1, W2, b2):
    nc_pad = 832
    rel = jnp.zeros((nc_pad * K, 3), relative.dtype).at[:relative.shape[0]].set(relative)
    grid = nc_pad // _CB
    return pl.pallas_call(
        _enc1_body,
        grid=(grid,),
        in_specs=[
            pl.BlockSpec((_CB * K, 3), lambda i: (i, 0)),
            pl.BlockSpec((3, 64), lambda i: (0, 0)),
            pl.BlockSpec((1, 64), lambda i: (0, 0)),
            pl.BlockSpec((64, 128), lambda i: (0, 0)),
            pl.BlockSpec((1, 128), lambda i: (0, 0)),
        ],
        out_specs=pl.BlockSpec((_CB, 128), lambda i: (i, 0)),
        out_shape=jax.ShapeDtypeStruct((nc_pad, 128), jnp.float32),
    )(rel, W1, b1, W2, b2)


def _enc2_body(rel_ref, fg_ref, w3a_ref, w3b_ref, b3_ref, w4_ref, b4_ref,
               out_ref):
    h = jnp.dot(rel_ref[...], w3a_ref[...], precision=_PREC)
    h = h + jnp.dot(fg_ref[...], w3b_ref[...], precision=_PREC)
    h = jnp.maximum(h + b3_ref[...], 0.0)
    h = jnp.maximum(jnp.dot(h, w4_ref[...], precision=_PREC)
                    + b4_ref[...], 0.0)
    for c in range(N2):
        out_ref[pl.ds(c, 1), :] = jnp.max(h[c * K:(c + 1) * K, :],
                                          axis=0, keepdims=True)


def _encoder2(relative2, fg, W3a, W3b, b3, W4, b4):
    return pl.pallas_call(
        _enc2_body,
        out_shape=jax.ShapeDtypeStruct((N2, 256), jnp.float32),
    )(relative2, fg, W3a, W3b, b3, W4, b4)


def _dec_body(n_valid, enc_ref, feats_ref, wd1a_ref, wd1b_ref, bd1_ref,
              wd2_ref, bd2_ref, bd3m_ref, bd3_ref, bd4m_ref, bd4_ref,
              o60_ref, o1200_ref):
    feats = feats_ref[...]
    rows = jax.lax.broadcasted_iota(jnp.int32, feats.shape, 0)
    gfeat = jnp.max(jnp.where(rows < n_valid, feats, -jnp.inf),
                    axis=0, keepdims=True)
    d1 = jnp.dot(enc_ref[...], wd1a_ref[...], precision=_PREC)
    d1 = d1 + jnp.dot(gfeat, wd1b_ref[...], precision=_PREC)
    d1 = jnp.maximum(d1 + bd1_ref[...], 0.0)
    mid = jnp.dot(d1, wd2_ref[...], precision=_PREC) + bd2_ref[...]
    o60_ref[...] = jnp.dot(mid, bd3m_ref[...], precision=_PREC) + bd3_ref[...]
    o1200_ref[...] = jnp.dot(mid, bd4m_ref[...], precision=_PREC) + bd4_ref[...]


def _decoder(encoding, feats, Wd1a, Wd1b, bd1, Wd2, bd2, bd3m, bd3t,
             bd4m, bd4t):
    return pl.pallas_call(
        functools.partial(_dec_body, N1),
        out_shape=[
            jax.ShapeDtypeStruct((N2, 60), jnp.float32),
            jax.ShapeDtypeStruct((N2, 1200), jnp.float32),
        ],
    )(encoding, feats, Wd1a, Wd1b, bd1, Wd2, bd2, bd3m, bd3t, bd4m, bd4t)
